# Initial kernel scaffold; baseline (speedup 1.0000x reference)
import jax
import jax.numpy as jnp
from jax.experimental import pallas as pl


def kernel(z, edge_index, W1, b1, W2, b2, W3, b3, conv_bias):
    raise NotImplementedError("write your pallas kernel here")


# speedup vs baseline: 1.9522x; 1.9522x over previous
import functools

import jax
import jax.numpy as jnp
from jax import lax
from jax.experimental import pallas as pl
from jax.experimental.pallas import tpu as pltpu
from jax.experimental.pallas import tpu_sc as plsc

N_NODES = 10000
N_EDGES = 320000
HID = 128
IN_CH = 2560

NC = 2
NS = 16
L = 16

CH_ROWS = 4
CHUNKS = 40
ROWS_PER_WORKER = CH_ROWS * CHUNKS
R2D = NS * ROWS_PER_WORKER
E_PAD = R2D * 128
NODES_PER_CORE = 5120
ACC_ROWS = 5376
TRASH_ROW = 5248
ROWS_PER_SUB = ACC_ROWS // NS


def _mlp_body(z_ref, w1_ref, b1_ref, w2_ref, b2_ref, w3_ref, b3_ref, out_ref):
    h = jnp.dot(z_ref[...], w1_ref[...], preferred_element_type=jnp.float32)
    h = jnp.maximum(h + b1_ref[...], 0.0)
    h = jnp.dot(h, w2_ref[...], preferred_element_type=jnp.float32)
    h = jnp.maximum(h + b2_ref[...], 0.0)
    h = jnp.dot(h, w3_ref[...], preferred_element_type=jnp.float32)
    out_ref[...] = h + b3_ref[...]


def _mlp(z, W1, b1, W2, b2, W3, b3):
    B = 1000
    grid = (N_NODES // B,)
    return pl.pallas_call(
        _mlp_body,
        grid=grid,
        in_specs=[
            pl.BlockSpec((B, IN_CH), lambda i: (i, 0)),
            pl.BlockSpec((IN_CH, HID), lambda i: (0, 0)),
            pl.BlockSpec((1, HID), lambda i: (0, 0)),
            pl.BlockSpec((HID, HID), lambda i: (0, 0)),
            pl.BlockSpec((1, HID), lambda i: (0, 0)),
            pl.BlockSpec((HID, HID), lambda i: (0, 0)),
            pl.BlockSpec((1, HID), lambda i: (0, 0)),
        ],
        out_specs=pl.BlockSpec((B, HID), lambda i: (i, 0)),
        out_shape=jax.ShapeDtypeStruct((N_NODES, HID), jnp.float32),
    )(z, W1, b1.reshape(1, HID), W2, b2.reshape(1, HID), W3, b3.reshape(1, HID))


def _sc_body(h_hbm, src_hbm, dst_hbm, part_hbm,
             src_v, dst_v, loc_v, rows_v, zbuf_v, sem,
             acc_sh):
    cid = lax.axis_index("c")
    sid = lax.axis_index("s")

    zero16 = jnp.zeros((L,), jnp.float32)
    base_node = cid * NODES_PER_CORE

    def zrow(i, carry):
        for j in range(HID // L):
            zbuf_v[i, pl.ds(j * L, L)] = zero16
        return carry
    lax.fori_loop(0, 48, zrow, 0)

    r_own = sid * ROWS_PER_SUB

    def zacc(i, carry):
        pltpu.sync_copy(zbuf_v, acc_sh.at[pl.ds(r_own + i * 48, 48)])
        return carry
    lax.fori_loop(0, ROWS_PER_SUB // 48, zacc, 0)

    plsc.subcore_barrier()

    base = sid * ROWS_PER_WORKER

    def chunk(c, carry):
        r0 = base + c * CH_ROWS
        pltpu.sync_copy(src_hbm.at[pl.ds(r0, CH_ROWS)], src_v)
        pltpu.sync_copy(dst_hbm.at[pl.ds(r0, CH_ROWS)], dst_v)
        descs = []
        for j in range(CH_ROWS):
            descs.append(pltpu.async_copy(
                h_hbm.at[src_v.at[j]],
                rows_v.at[pl.ds(j * 128, 128)], sem))
        for j in range(CH_ROWS):
            for k in range(128 // L):
                d = dst_v[j, pl.ds(k * L, L)]
                loc = d - base_node
                bad = (loc < 0) | (loc >= NODES_PER_CORE)
                loc = jnp.where(bad, TRASH_ROW, loc)
                loc_v[j, pl.ds(k * L, L)] = loc
        for d in descs:
            d.wait()
        for j in range(CH_ROWS):
            pltpu.sync_copy(rows_v.at[pl.ds(j * 128, 128)],
                            acc_sh.at[loc_v.at[j]], add=True)
        return carry
    lax.fori_loop(0, CHUNKS, chunk, 0)

    plsc.subcore_barrier()

    pltpu.sync_copy(acc_sh.at[pl.ds(r_own, ROWS_PER_SUB)],
                    part_hbm.at[cid, pl.ds(r_own, ROWS_PER_SUB)])


def _sc_scatter(h, src2d, dst2d):
    mesh = plsc.VectorSubcoreMesh(core_axis_name="c", subcore_axis_name="s")
    fn = functools.partial(
        pl.kernel,
        out_type=jax.ShapeDtypeStruct((NC, ACC_ROWS, HID), jnp.float32),
        mesh=mesh,
        scratch_types=[
            pltpu.VMEM((CH_ROWS, 128), jnp.int32),
            pltpu.VMEM((CH_ROWS, 128), jnp.int32),
            pltpu.VMEM((CH_ROWS, 128), jnp.int32),
            pltpu.VMEM((CH_ROWS * 128, HID), jnp.float32),
            pltpu.VMEM((48, HID), jnp.float32),
            pltpu.SemaphoreType.DMA,
            pltpu.VMEM_SHARED((ACC_ROWS, HID), jnp.float32),
        ],
    )(_sc_body)
    return fn(h, src2d, dst2d)


def _sc_cnt_body(dst_hbm, cntp_hbm,
                 dst_v, loc_v, ones_v, zcnt_v, cnt_sh):
    cid = lax.axis_index("c")
    sid = lax.axis_index("s")

    zero16 = jnp.zeros((L,), jnp.float32)
    one16 = jnp.ones((L,), jnp.float32)
    base_node = cid * NODES_PER_CORE

    def zcrow(i, carry):
        for j in range(128 // L):
            zcnt_v[i, pl.ds(j * L, L)] = zero16
        return carry
    lax.fori_loop(0, 48, zcrow, 0)

    def orow(i, carry):
        for j in range(128 // L):
            ones_v[i, pl.ds(j * L, L)] = one16
        return carry
    lax.fori_loop(0, 128, orow, 0)

    r_own = sid * ROWS_PER_SUB

    def zc(i, carry):
        pltpu.sync_copy(zcnt_v, cnt_sh.at[pl.ds(r_own + i * 48, 48)])
        return carry
    lax.fori_loop(0, ROWS_PER_SUB // 48, zc, 0)

    plsc.subcore_barrier()

    base = sid * ROWS_PER_WORKER

    def chunk(c, carry):
        r0 = base + c * CH_ROWS
        pltpu.sync_copy(dst_hbm.at[pl.ds(r0, CH_ROWS)], dst_v)
        for j in range(CH_ROWS):
            for k in range(128 // L):
                d = dst_v[j, pl.ds(k * L, L)]
                loc = d - base_node
                bad = (loc < 0) | (loc >= NODES_PER_CORE)
                loc_v[j, pl.ds(k * L, L)] = jnp.where(bad, TRASH_ROW, loc)
        for j in range(CH_ROWS):
            pltpu.sync_copy(ones_v, cnt_sh.at[loc_v.at[j]], add=True)
        return carry
    lax.fori_loop(0, CHUNKS, chunk, 0)

    plsc.subcore_barrier()

    pltpu.sync_copy(cnt_sh.at[pl.ds(r_own, ROWS_PER_SUB)],
                    cntp_hbm.at[cid, pl.ds(r_own, ROWS_PER_SUB)])


def _sc_counts(dst2d):
    mesh = plsc.VectorSubcoreMesh(core_axis_name="c", subcore_axis_name="s")
    fn = functools.partial(
        pl.kernel,
        out_type=jax.ShapeDtypeStruct((NC, ACC_ROWS, 128), jnp.float32),
        mesh=mesh,
        scratch_types=[
            pltpu.VMEM((CH_ROWS, 128), jnp.int32),
            pltpu.VMEM((CH_ROWS, 128), jnp.int32),
            pltpu.VMEM((128, 128), jnp.float32),
            pltpu.VMEM((48, 128), jnp.float32),
            pltpu.VMEM_SHARED((ACC_ROWS, 128), jnp.float32),
        ],
    )(_sc_cnt_body)
    return fn(dst2d)


def _fin_body(part_ref, cnt_ref, bias_ref, out_ref):
    lo = part_ref[0, :NODES_PER_CORE]
    hi = part_ref[1, :N_NODES - NODES_PER_CORE]
    s = jnp.concatenate([lo, hi], axis=0)
    clo = cnt_ref[0, :NODES_PER_CORE, 0:1]
    chi = cnt_ref[1, :N_NODES - NODES_PER_CORE, 0:1]
    c = jnp.maximum(jnp.concatenate([clo, chi], axis=0), 1.0)
    out_ref[...] = s / c + bias_ref[...]


def _finalize(part, cntp, conv_bias):
    return pl.pallas_call(
        _fin_body,
        out_shape=jax.ShapeDtypeStruct((N_NODES, HID), jnp.float32),
    )(part, cntp, conv_bias.reshape(1, HID))


def kernel(z, edge_index, W1, b1, W2, b2, W3, b3, conv_bias):
    h = _mlp(z, W1, b1, W2, b2, W3, b3)
    src = edge_index[0]
    dst = edge_index[1]
    pad = E_PAD - N_EDGES
    src_p = jnp.concatenate([src, jnp.zeros((pad,), jnp.int32)])
    dst_p = jnp.concatenate([dst, jnp.full((pad,), N_NODES, jnp.int32)])
    src2d = src_p.reshape(R2D, 128)
    dst2d = dst_p.reshape(R2D, 128)
    cntp = _sc_counts(dst2d)
    part = _sc_scatter(h, src2d, dst2d)
    return _finalize(part, cntp, conv_bias)


# speedup vs baseline: 2.2443x; 1.1497x over previous
import functools

import jax
import jax.numpy as jnp
from jax import lax
from jax.experimental import pallas as pl
from jax.experimental.pallas import tpu as pltpu
from jax.experimental.pallas import tpu_sc as plsc

N_NODES = 10000
N_EDGES = 320000
HID = 128
IN_CH = 2560

NC = 2
NS = 16
L = 16

CH_ROWS = 4
CHUNKS = 40
ROWS_PER_WORKER = 160
R2D = NS * ROWS_PER_WORKER
E_PAD = R2D * 128
NODES_PER_CORE = 5120
ACC_ROWS = 5376
TRASH_ROW = 5248
ROWS_PER_SUB = ACC_ROWS // NS

GCHUNKS = ROWS_PER_WORKER
GPAIRS = GCHUNKS // 2


def _mlp_body(z_ref, w1_ref, b1_ref, w2_ref, b2_ref, w3_ref, b3_ref, out_ref):
    h = jnp.dot(z_ref[...], w1_ref[...], preferred_element_type=jnp.float32)
    h = jnp.maximum(h + b1_ref[...], 0.0)
    h = jnp.dot(h, w2_ref[...], preferred_element_type=jnp.float32)
    h = jnp.maximum(h + b2_ref[...], 0.0)
    h = jnp.dot(h, w3_ref[...], preferred_element_type=jnp.float32)
    out_ref[...] = h + b3_ref[...]


def _mlp(z, W1, b1, W2, b2, W3, b3):
    B = 1000
    grid = (N_NODES // B,)
    return pl.pallas_call(
        _mlp_body,
        grid=grid,
        in_specs=[
            pl.BlockSpec((B, IN_CH), lambda i: (i, 0)),
            pl.BlockSpec((IN_CH, HID), lambda i: (0, 0)),
            pl.BlockSpec((1, HID), lambda i: (0, 0)),
            pl.BlockSpec((HID, HID), lambda i: (0, 0)),
            pl.BlockSpec((1, HID), lambda i: (0, 0)),
            pl.BlockSpec((HID, HID), lambda i: (0, 0)),
            pl.BlockSpec((1, HID), lambda i: (0, 0)),
        ],
        out_specs=pl.BlockSpec((B, HID), lambda i: (i, 0)),
        out_shape=jax.ShapeDtypeStruct((N_NODES, HID), jnp.float32),
    )(z, W1, b1.reshape(1, HID), W2, b2.reshape(1, HID), W3, b3.reshape(1, HID))


def _sc_body(h_hbm, src_hbm, dst_hbm, part_hbm,
             src0, src1, dst0, dst1, rows0, rows1, loc0, loc1, zbuf_v,
             sem_g, sem_s, sem_i, acc_sh):
    cid = lax.axis_index("c")
    sid = lax.axis_index("s")

    zero16 = jnp.zeros((L,), jnp.float32)
    base_node = cid * NODES_PER_CORE

    def zrow(i, carry):
        for j in range(HID // L):
            zbuf_v[i, pl.ds(j * L, L)] = zero16
        return carry
    lax.fori_loop(0, 48, zrow, 0)

    r_own = sid * ROWS_PER_SUB

    def zacc(i, carry):
        pltpu.sync_copy(zbuf_v, acc_sh.at[pl.ds(r_own + i * 48, 48)])
        return carry
    lax.fori_loop(0, ROWS_PER_SUB // 48, zacc, 0)

    plsc.subcore_barrier()

    base = sid * ROWS_PER_WORKER
    srcs = (src0, src1)
    dsts = (dst0, dst1)
    rows = (rows0, rows1)
    locs = (loc0, loc1)

    def fire_idx(c, cur):
        pltpu.async_copy(src_hbm.at[pl.ds(base + c, 1)], srcs[cur], sem_i)
        pltpu.async_copy(dst_hbm.at[pl.ds(base + c, 1)], dsts[cur], sem_i)

    def wait_idx(cur):
        pltpu.make_async_copy(src_hbm.at[pl.ds(base, 1)], srcs[cur],
                              sem_i).wait()
        pltpu.make_async_copy(dst_hbm.at[pl.ds(base, 1)], dsts[cur],
                              sem_i).wait()

    def fire_gather(cur):
        pltpu.async_copy(h_hbm.at[srcs[cur].at[0]], rows[cur], sem_g)

    def wait_gather(cur):
        pltpu.make_async_copy(h_hbm.at[srcs[cur].at[0]], rows[cur],
                              sem_g).wait()

    def remap(cur):
        for k in range(128 // L):
            d = dsts[cur][0, pl.ds(k * L, L)]
            loc = d - base_node
            bad = (loc < 0) | (loc >= NODES_PER_CORE)
            loc = jnp.where(bad, TRASH_ROW + (d & 127), loc)
            locs[cur][0, pl.ds(k * L, L)] = loc

    def fire_scatter(cur):
        pltpu.async_copy(rows[cur], acc_sh.at[locs[cur].at[0]], sem_s,
                         add=True)

    def wait_scatter(cur):
        pltpu.make_async_copy(rows[cur], acc_sh.at[locs[cur].at[0]],
                              sem_s).wait()

    pltpu.sync_copy(src_hbm.at[pl.ds(base, 1)], src0)
    pltpu.sync_copy(dst_hbm.at[pl.ds(base, 1)], dst0)
    fire_idx(1, 1)
    fire_gather(0)

    def pair(p, carry):
        for cur in range(2):
            c = 2 * p + cur
            other = 1 - cur
            wait_gather(cur)
            remap(cur)
            if cur == 0:
                @pl.when(p > 0)
                def _():
                    wait_scatter(other)
            else:
                wait_scatter(other)
            fire_scatter(cur)
            if cur == 0:
                wait_idx(other)
                fire_gather(other)
            else:
                @pl.when(p < GPAIRS - 1)
                def _():
                    wait_idx(other)
                    fire_gather(other)

            @pl.when(c + 2 < GCHUNKS)
            def _():
                fire_idx(c + 2, cur)
        return carry
    lax.fori_loop(0, GPAIRS, pair, 0)
    wait_scatter(1)

    plsc.subcore_barrier()

    pltpu.sync_copy(acc_sh.at[pl.ds(r_own, ROWS_PER_SUB)],
                    part_hbm.at[cid, pl.ds(r_own, ROWS_PER_SUB)])


def _sc_scatter(h, src2d, dst2d):
    mesh = plsc.VectorSubcoreMesh(core_axis_name="c", subcore_axis_name="s")
    fn = functools.partial(
        pl.kernel,
        out_type=jax.ShapeDtypeStruct((NC, ACC_ROWS, HID), jnp.float32),
        mesh=mesh,
        scratch_types=[
            pltpu.VMEM((1, 128), jnp.int32),
            pltpu.VMEM((1, 128), jnp.int32),
            pltpu.VMEM((1, 128), jnp.int32),
            pltpu.VMEM((1, 128), jnp.int32),
            pltpu.VMEM((128, HID), jnp.float32),
            pltpu.VMEM((128, HID), jnp.float32),
            pltpu.VMEM((1, 128), jnp.int32),
            pltpu.VMEM((1, 128), jnp.int32),
            pltpu.VMEM((48, HID), jnp.float32),
            pltpu.SemaphoreType.DMA,
            pltpu.SemaphoreType.DMA,
            pltpu.SemaphoreType.DMA,
            pltpu.VMEM_SHARED((ACC_ROWS, HID), jnp.float32),
        ],
    )(_sc_body)
    return fn(h, src2d, dst2d)


def _sc_cnt_body(dst_hbm, cntp_hbm,
                 dst_v, loc_v, ones_v, zcnt_v, cnt_sh):
    cid = lax.axis_index("c")
    sid = lax.axis_index("s")

    zero16 = jnp.zeros((L,), jnp.float32)
    one16 = jnp.ones((L,), jnp.float32)
    base_node = cid * NODES_PER_CORE

    def zcrow(i, carry):
        for j in range(128 // L):
            zcnt_v[i, pl.ds(j * L, L)] = zero16
        return carry
    lax.fori_loop(0, 48, zcrow, 0)

    def orow(i, carry):
        for j in range(128 // L):
            ones_v[i, pl.ds(j * L, L)] = one16
        return carry
    lax.fori_loop(0, 128, orow, 0)

    r_own = sid * ROWS_PER_SUB

    def zc(i, carry):
        pltpu.sync_copy(zcnt_v, cnt_sh.at[pl.ds(r_own + i * 48, 48)])
        return carry
    lax.fori_loop(0, ROWS_PER_SUB // 48, zc, 0)

    plsc.subcore_barrier()

    base = sid * ROWS_PER_WORKER

    def chunk(c, carry):
        r0 = base + c * CH_ROWS
        pltpu.sync_copy(dst_hbm.at[pl.ds(r0, CH_ROWS)], dst_v)
        for j in range(CH_ROWS):
            for k in range(128 // L):
                d = dst_v[j, pl.ds(k * L, L)]
                loc = d - base_node
                bad = (loc < 0) | (loc >= NODES_PER_CORE)
                loc_v[j, pl.ds(k * L, L)] = jnp.where(
                    bad, TRASH_ROW + (d & 127), loc)
        for j in range(CH_ROWS):
            pltpu.sync_copy(ones_v, cnt_sh.at[loc_v.at[j]], add=True)
        return carry
    lax.fori_loop(0, CHUNKS, chunk, 0)

    plsc.subcore_barrier()

    pltpu.sync_copy(cnt_sh.at[pl.ds(r_own, ROWS_PER_SUB)],
                    cntp_hbm.at[cid, pl.ds(r_own, ROWS_PER_SUB)])


def _sc_counts(dst2d):
    mesh = plsc.VectorSubcoreMesh(core_axis_name="c", subcore_axis_name="s")
    fn = functools.partial(
        pl.kernel,
        out_type=jax.ShapeDtypeStruct((NC, ACC_ROWS, 128), jnp.float32),
        mesh=mesh,
        scratch_types=[
            pltpu.VMEM((CH_ROWS, 128), jnp.int32),
            pltpu.VMEM((CH_ROWS, 128), jnp.int32),
            pltpu.VMEM((128, 128), jnp.float32),
            pltpu.VMEM((48, 128), jnp.float32),
            pltpu.VMEM_SHARED((ACC_ROWS, 128), jnp.float32),
        ],
    )(_sc_cnt_body)
    return fn(dst2d)


def _fin_body(part_ref, cnt_ref, bias_ref, out_ref):
    lo = part_ref[0, :NODES_PER_CORE]
    hi = part_ref[1, :N_NODES - NODES_PER_CORE]
    s = jnp.concatenate([lo, hi], axis=0)
    clo = cnt_ref[0, :NODES_PER_CORE, 0:1]
    chi = cnt_ref[1, :N_NODES - NODES_PER_CORE, 0:1]
    c = jnp.maximum(jnp.concatenate([clo, chi], axis=0), 1.0)
    out_ref[...] = s / c + bias_ref[...]


def _finalize(part, cntp, conv_bias):
    return pl.pallas_call(
        _fin_body,
        out_shape=jax.ShapeDtypeStruct((N_NODES, HID), jnp.float32),
    )(part, cntp, conv_bias.reshape(1, HID))


def kernel(z, edge_index, W1, b1, W2, b2, W3, b3, conv_bias):
    h = _mlp(z, W1, b1, W2, b2, W3, b3)
    src = edge_index[0]
    dst = edge_index[1]
    pad = E_PAD - N_EDGES
    src_p = jnp.concatenate([src, jnp.zeros((pad,), jnp.int32)])
    dst_p = jnp.concatenate([dst, jnp.full((pad,), N_NODES, jnp.int32)])
    src2d = src_p.reshape(R2D, 128)
    dst2d = dst_p.reshape(R2D, 128)
    cntp = _sc_counts(dst2d)
    part = _sc_scatter(h, src2d, dst2d)
    return _finalize(part, cntp, conv_bias)


# speedup vs baseline: 2.2839x; 1.0176x over previous
import functools

import jax
import jax.numpy as jnp
from jax import lax
from jax.experimental import pallas as pl
from jax.experimental.pallas import tpu as pltpu
from jax.experimental.pallas import tpu_sc as plsc

N_NODES = 10000
N_EDGES = 320000
HID = 128
IN_CH = 2560

NC = 2
NS = 16
L = 16

CH_ROWS = 4
CHUNKS = 40
ROWS_PER_WORKER = 160
R2D = NS * ROWS_PER_WORKER
E_PAD = R2D * 128
NODES_PER_CORE = 5120
ACC_ROWS = 5376
TRASH_ROW = 5248
ROWS_PER_SUB = ACC_ROWS // NS

GCHUNKS = ROWS_PER_WORKER
GPAIRS = GCHUNKS // 2


def _mlp_body(z_ref, w1_ref, b1_ref, w2_ref, b2_ref, w3_ref, b3_ref, out_ref):
    h = jnp.dot(z_ref[...], w1_ref[...], preferred_element_type=jnp.float32)
    h = jnp.maximum(h + b1_ref[...], 0.0)
    h = jnp.dot(h, w2_ref[...], preferred_element_type=jnp.float32)
    h = jnp.maximum(h + b2_ref[...], 0.0)
    h = jnp.dot(h, w3_ref[...], preferred_element_type=jnp.float32)
    out_ref[...] = h + b3_ref[...]


def _mlp(z, W1, b1, W2, b2, W3, b3):
    B = 1000
    grid = (N_NODES // B,)
    return pl.pallas_call(
        _mlp_body,
        grid=grid,
        in_specs=[
            pl.BlockSpec((B, IN_CH), lambda i: (i, 0)),
            pl.BlockSpec((IN_CH, HID), lambda i: (0, 0)),
            pl.BlockSpec((1, HID), lambda i: (0, 0)),
            pl.BlockSpec((HID, HID), lambda i: (0, 0)),
            pl.BlockSpec((1, HID), lambda i: (0, 0)),
            pl.BlockSpec((HID, HID), lambda i: (0, 0)),
            pl.BlockSpec((1, HID), lambda i: (0, 0)),
        ],
        out_specs=pl.BlockSpec((B, HID), lambda i: (i, 0)),
        out_shape=jax.ShapeDtypeStruct((N_NODES, HID), jnp.float32),
    )(z, W1, b1.reshape(1, HID), W2, b2.reshape(1, HID), W3, b3.reshape(1, HID))


def _sc_body(h_hbm, src_hbm, dst_hbm, part_hbm,
             src0, src1, dst0, dst1, rows0, rows1, loc0, loc1, zbuf_v,
             sem_g, sem_s, sem_i, acc_sh):
    cid = lax.axis_index("c")
    sid = lax.axis_index("s")

    zero16 = jnp.zeros((L,), jnp.float32)
    base_node = cid * NODES_PER_CORE

    def zrow(i, carry):
        for j in range(HID // L):
            zbuf_v[i, pl.ds(j * L, L)] = zero16
        return carry
    lax.fori_loop(0, 48, zrow, 0)

    r_own = sid * ROWS_PER_SUB

    def zacc(i, carry):
        pltpu.sync_copy(zbuf_v, acc_sh.at[pl.ds(r_own + i * 48, 48)])
        return carry
    lax.fori_loop(0, ROWS_PER_SUB // 48, zacc, 0)

    plsc.subcore_barrier()

    base = sid * ROWS_PER_WORKER
    srcs = (src0, src1)
    dsts = (dst0, dst1)
    rows = (rows0, rows1)
    locs = (loc0, loc1)

    def fire_idx(c, cur):
        pltpu.async_copy(src_hbm.at[pl.ds(base + c, 1)], srcs[cur], sem_i)
        pltpu.async_copy(dst_hbm.at[pl.ds(base + c, 1)], dsts[cur], sem_i)

    def wait_idx(cur):
        pltpu.make_async_copy(src_hbm.at[pl.ds(base, 1)], srcs[cur],
                              sem_i).wait()
        pltpu.make_async_copy(dst_hbm.at[pl.ds(base, 1)], dsts[cur],
                              sem_i).wait()

    def fire_gather(cur):
        pltpu.async_copy(h_hbm.at[srcs[cur].at[0]], rows[cur], sem_g)

    def wait_gather(cur):
        pltpu.make_async_copy(h_hbm.at[srcs[cur].at[0]], rows[cur],
                              sem_g).wait()

    def remap(cur):
        for k in range(128 // L):
            d = dsts[cur][0, pl.ds(k * L, L)]
            loc = d - base_node
            bad = (loc < 0) | (loc >= NODES_PER_CORE)
            loc = jnp.where(bad, TRASH_ROW + (d & 127), loc)
            locs[cur][0, pl.ds(k * L, L)] = loc

    def fire_scatter(cur):
        pltpu.async_copy(rows[cur], acc_sh.at[locs[cur].at[0]], sem_s,
                         add=True)

    def wait_scatter(cur):
        pltpu.make_async_copy(rows[cur], acc_sh.at[locs[cur].at[0]],
                              sem_s).wait()

    pltpu.sync_copy(src_hbm.at[pl.ds(base, 1)], src0)
    pltpu.sync_copy(dst_hbm.at[pl.ds(base, 1)], dst0)
    fire_idx(1, 1)
    fire_gather(0)

    def pair(p, carry):
        for cur in range(2):
            c = 2 * p + cur
            other = 1 - cur
            wait_gather(cur)
            remap(cur)
            if cur == 0:
                @pl.when(p > 0)
                def _():
                    wait_scatter(other)
            else:
                wait_scatter(other)
            fire_scatter(cur)
            if cur == 0:
                wait_idx(other)
                fire_gather(other)
            else:
                @pl.when(p < GPAIRS - 1)
                def _():
                    wait_idx(other)
                    fire_gather(other)

            @pl.when(c + 2 < GCHUNKS)
            def _():
                fire_idx(c + 2, cur)
        return carry
    lax.fori_loop(0, GPAIRS, pair, 0)
    wait_scatter(1)

    plsc.subcore_barrier()

    pltpu.sync_copy(acc_sh.at[pl.ds(r_own, ROWS_PER_SUB)],
                    part_hbm.at[cid, pl.ds(r_own, ROWS_PER_SUB)])


def _sc_scatter(h, src2d, dst2d):
    mesh = plsc.VectorSubcoreMesh(core_axis_name="c", subcore_axis_name="s")
    fn = functools.partial(
        pl.kernel,
        out_type=jax.ShapeDtypeStruct((NC, ACC_ROWS, HID), jnp.float32),
        mesh=mesh,
        scratch_types=[
            pltpu.VMEM((1, 128), jnp.int32),
            pltpu.VMEM((1, 128), jnp.int32),
            pltpu.VMEM((1, 128), jnp.int32),
            pltpu.VMEM((1, 128), jnp.int32),
            pltpu.VMEM((128, HID), jnp.float32),
            pltpu.VMEM((128, HID), jnp.float32),
            pltpu.VMEM((1, 128), jnp.int32),
            pltpu.VMEM((1, 128), jnp.int32),
            pltpu.VMEM((48, HID), jnp.float32),
            pltpu.SemaphoreType.DMA,
            pltpu.SemaphoreType.DMA,
            pltpu.SemaphoreType.DMA,
            pltpu.VMEM_SHARED((ACC_ROWS, HID), jnp.float32),
        ],
    )(_sc_body)
    return fn(h, src2d, dst2d)


def _sc_cnt_body(dst_hbm, cntp_hbm,
                 dst0, dst1, loc0, loc1, ones_v, zcnt_v,
                 sem_s, sem_i0, sem_i1, cnt_sh):
    cid = lax.axis_index("c")
    sid = lax.axis_index("s")

    zero16 = jnp.zeros((L,), jnp.float32)
    one16 = jnp.ones((L,), jnp.float32)
    base_node = cid * NODES_PER_CORE

    def zcrow(i, carry):
        for j in range(128 // L):
            zcnt_v[i, pl.ds(j * L, L)] = zero16
        return carry
    lax.fori_loop(0, 48, zcrow, 0)

    def orow(i, carry):
        for j in range(128 // L):
            ones_v[i, pl.ds(j * L, L)] = one16
        return carry
    lax.fori_loop(0, 128, orow, 0)

    r_own = sid * ROWS_PER_SUB

    def zc(i, carry):
        pltpu.sync_copy(zcnt_v, cnt_sh.at[pl.ds(r_own + i * 48, 48)])
        return carry
    lax.fori_loop(0, ROWS_PER_SUB // 48, zc, 0)

    plsc.subcore_barrier()

    base = sid * ROWS_PER_WORKER
    dsts = (dst0, dst1)
    locs = (loc0, loc1)
    sems = (sem_i0, sem_i1)

    def fire_idx(c, cur):
        pltpu.async_copy(dst_hbm.at[pl.ds(base + c * CH_ROWS, CH_ROWS)],
                         dsts[cur], sems[cur])

    def wait_idx(cur):
        pltpu.make_async_copy(dst_hbm.at[pl.ds(base, CH_ROWS)],
                              dsts[cur], sems[cur]).wait()

    def remap(cur):
        for j in range(CH_ROWS):
            for k in range(128 // L):
                d = dsts[cur][j, pl.ds(k * L, L)]
                loc = d - base_node
                bad = (loc < 0) | (loc >= NODES_PER_CORE)
                locs[cur][j, pl.ds(k * L, L)] = jnp.where(
                    bad, TRASH_ROW + (d & 127), loc)

    def fire_scatter(cur):
        for j in range(CH_ROWS):
            pltpu.async_copy(ones_v, cnt_sh.at[locs[cur].at[j]], sem_s,
                             add=True)

    def wait_scatter(cur):
        for j in range(CH_ROWS):
            pltpu.make_async_copy(ones_v, cnt_sh.at[locs[cur].at[j]],
                                  sem_s).wait()

    pltpu.sync_copy(dst_hbm.at[pl.ds(base, CH_ROWS)], dst0)
    fire_idx(1, 1)

    def pair(p, carry):
        for cur in range(2):
            c = 2 * p + cur
            other = 1 - cur
            if cur == 0:
                @pl.when(p > 0)
                def _():
                    wait_idx(0)
            else:
                wait_idx(1)
            remap(cur)
            if cur == 0:
                @pl.when(p > 0)
                def _():
                    wait_scatter(other)
            else:
                wait_scatter(other)
            fire_scatter(cur)

            @pl.when(c + 2 < CHUNKS)
            def _():
                fire_idx(c + 2, cur)
        return carry
    lax.fori_loop(0, CHUNKS // 2, pair, 0)
    wait_scatter(1)

    plsc.subcore_barrier()

    pltpu.sync_copy(cnt_sh.at[pl.ds(r_own, ROWS_PER_SUB)],
                    cntp_hbm.at[cid, pl.ds(r_own, ROWS_PER_SUB)])


def _sc_counts(dst2d):
    mesh = plsc.VectorSubcoreMesh(core_axis_name="c", subcore_axis_name="s")
    fn = functools.partial(
        pl.kernel,
        out_type=jax.ShapeDtypeStruct((NC, ACC_ROWS, 128), jnp.float32),
        mesh=mesh,
        scratch_types=[
            pltpu.VMEM((CH_ROWS, 128), jnp.int32),
            pltpu.VMEM((CH_ROWS, 128), jnp.int32),
            pltpu.VMEM((CH_ROWS, 128), jnp.int32),
            pltpu.VMEM((CH_ROWS, 128), jnp.int32),
            pltpu.VMEM((128, 128), jnp.float32),
            pltpu.VMEM((48, 128), jnp.float32),
            pltpu.SemaphoreType.DMA,
            pltpu.SemaphoreType.DMA,
            pltpu.SemaphoreType.DMA,
            pltpu.VMEM_SHARED((ACC_ROWS, 128), jnp.float32),
        ],
    )(_sc_cnt_body)
    return fn(dst2d)


def _fin_body(part_ref, cnt_ref, bias_ref, out_ref):
    lo = part_ref[0, :NODES_PER_CORE]
    hi = part_ref[1, :N_NODES - NODES_PER_CORE]
    s = jnp.concatenate([lo, hi], axis=0)
    clo = cnt_ref[0, :NODES_PER_CORE, 0:1]
    chi = cnt_ref[1, :N_NODES - NODES_PER_CORE, 0:1]
    c = jnp.maximum(jnp.concatenate([clo, chi], axis=0), 1.0)
    out_ref[...] = s / c + bias_ref[...]


def _finalize(part, cntp, conv_bias):
    return pl.pallas_call(
        _fin_body,
        out_shape=jax.ShapeDtypeStruct((N_NODES, HID), jnp.float32),
    )(part, cntp, conv_bias.reshape(1, HID))


def kernel(z, edge_index, W1, b1, W2, b2, W3, b3, conv_bias):
    src = edge_index[0]
    dst = edge_index[1]
    pad = E_PAD - N_EDGES
    src_p = jnp.concatenate([src, jnp.zeros((pad,), jnp.int32)])
    dst_p = jnp.concatenate([dst, jnp.full((pad,), N_NODES, jnp.int32)])
    src2d = src_p.reshape(R2D, 128)
    dst2d = dst_p.reshape(R2D, 128)
    cntp = _sc_counts(dst2d)
    h = _mlp(z, W1, b1, W2, b2, W3, b3)
    part = _sc_scatter(h, src2d, dst2d)
    return _finalize(part, cntp, conv_bias)


# speedup vs baseline: 2.3818x; 1.0428x over previous
import functools

import jax
import jax.numpy as jnp
from jax import lax
from jax.experimental import pallas as pl
from jax.experimental.pallas import tpu as pltpu
from jax.experimental.pallas import tpu_sc as plsc

N_NODES = 10000
N_EDGES = 320000
HID = 128
IN_CH = 2560

NC = 2
NS = 16
L = 16

CH_ROWS = 4
CHUNKS = 40
ROWS_PER_WORKER = 160
R2D = NS * ROWS_PER_WORKER
E_PAD = R2D * 128
NODES_PER_CORE = 5120
ACC_ROWS = 5376
TRASH_ROW = 5248
ROWS_PER_SUB = ACC_ROWS // NS

GCHUNKS = ROWS_PER_WORKER
GPAIRS = GCHUNKS // 2


def _mlp_body(z_ref, w1_ref, b1_ref, w2_ref, b2_ref, w3_ref, b3_ref, out_ref):
    h = jnp.dot(z_ref[...], w1_ref[...], preferred_element_type=jnp.float32)
    h = jnp.maximum(h + b1_ref[...], 0.0)
    h = jnp.dot(h, w2_ref[...], preferred_element_type=jnp.float32)
    h = jnp.maximum(h + b2_ref[...], 0.0)
    h = jnp.dot(h, w3_ref[...], preferred_element_type=jnp.float32)
    out_ref[...] = h + b3_ref[...]


def _mlp(z, W1, b1, W2, b2, W3, b3):
    B = 1000
    grid = (N_NODES // B,)
    return pl.pallas_call(
        _mlp_body,
        grid=grid,
        in_specs=[
            pl.BlockSpec((B, IN_CH), lambda i: (i, 0)),
            pl.BlockSpec((IN_CH, HID), lambda i: (0, 0)),
            pl.BlockSpec((1, HID), lambda i: (0, 0)),
            pl.BlockSpec((HID, HID), lambda i: (0, 0)),
            pl.BlockSpec((1, HID), lambda i: (0, 0)),
            pl.BlockSpec((HID, HID), lambda i: (0, 0)),
            pl.BlockSpec((1, HID), lambda i: (0, 0)),
        ],
        out_specs=pl.BlockSpec((B, HID), lambda i: (i, 0)),
        out_shape=jax.ShapeDtypeStruct((N_NODES, HID), jnp.float32),
    )(z, W1, b1.reshape(1, HID), W2, b2.reshape(1, HID), W3, b3.reshape(1, HID))


def _sc_body(h_hbm, src_hbm, dst_hbm, part_hbm,
             src0, src1, src2, src3, dst0, dst1, dst2, dst3,
             rows0, rows1, rows2, rows3, loc0, loc1, loc2, loc3, zbuf_v,
             sg0, sg1, sg2, sg3, ss0, ss1, ss2, ss3, si0, si1, si2, si3,
             acc_sh):
    cid = lax.axis_index("c")
    sid = lax.axis_index("s")

    zero16 = jnp.zeros((L,), jnp.float32)
    base_node = cid * NODES_PER_CORE

    def zrow(i, carry):
        for j in range(HID // L):
            zbuf_v[i, pl.ds(j * L, L)] = zero16
        return carry
    lax.fori_loop(0, 48, zrow, 0)

    r_own = sid * ROWS_PER_SUB

    def zacc(i, carry):
        pltpu.sync_copy(zbuf_v, acc_sh.at[pl.ds(r_own + i * 48, 48)])
        return carry
    lax.fori_loop(0, ROWS_PER_SUB // 48, zacc, 0)

    plsc.subcore_barrier()

    base = sid * ROWS_PER_WORKER
    srcs = (src0, src1, src2, src3)
    dsts = (dst0, dst1, dst2, dst3)
    rows = (rows0, rows1, rows2, rows3)
    locs = (loc0, loc1, loc2, loc3)
    sgs = (sg0, sg1, sg2, sg3)
    sss = (ss0, ss1, ss2, ss3)
    sis = (si0, si1, si2, si3)

    def fire_idx(c, m):
        pltpu.async_copy(src_hbm.at[pl.ds(base + c, 1)], srcs[m], sis[m])
        pltpu.async_copy(dst_hbm.at[pl.ds(base + c, 1)], dsts[m], sis[m])

    def wait_idx(m):
        pltpu.make_async_copy(src_hbm.at[pl.ds(base, 1)], srcs[m],
                              sis[m]).wait()
        pltpu.make_async_copy(dst_hbm.at[pl.ds(base, 1)], dsts[m],
                              sis[m]).wait()

    def fire_gather(m):
        pltpu.async_copy(h_hbm.at[srcs[m].at[0]], rows[m], sgs[m])

    def wait_gather(m):
        pltpu.make_async_copy(h_hbm.at[srcs[m].at[0]], rows[m],
                              sgs[m]).wait()

    def remap(m):
        for k in range(128 // L):
            d = dsts[m][0, pl.ds(k * L, L)]
            loc = d - base_node
            bad = (loc < 0) | (loc >= NODES_PER_CORE)
            loc = jnp.where(bad, TRASH_ROW + (d & 127), loc)
            locs[m][0, pl.ds(k * L, L)] = loc

    def fire_scatter(m):
        pltpu.async_copy(rows[m], acc_sh.at[locs[m].at[0]], sss[m],
                         add=True)

    def wait_scatter(m):
        pltpu.make_async_copy(rows[m], acc_sh.at[locs[m].at[0]],
                              sss[m]).wait()

    pltpu.sync_copy(src_hbm.at[pl.ds(base, 1)], src0)
    pltpu.sync_copy(dst_hbm.at[pl.ds(base, 1)], dst0)
    pltpu.sync_copy(src_hbm.at[pl.ds(base + 1, 1)], src1)
    pltpu.sync_copy(dst_hbm.at[pl.ds(base + 1, 1)], dst1)
    fire_idx(2, 2)
    fire_idx(3, 3)
    fire_gather(0)
    fire_gather(1)

    def quad(q, carry):
        for m in range(4):
            c = 4 * q + m
            if m < 2:
                @pl.when(q > 0)
                def _():
                    wait_scatter((m + 2) % 4)
            else:
                wait_scatter((m + 2) % 4)
            if m < 2:
                wait_idx((m + 2) % 4)
                fire_gather((m + 2) % 4)
            else:
                @pl.when(q < GCHUNKS // 4 - 1)
                def _():
                    wait_idx((m + 2) % 4)
                    fire_gather((m + 2) % 4)
            wait_gather(m)
            remap(m)

            @pl.when(q < GCHUNKS // 4 - 1)
            def _():
                fire_idx(c + 4, m)
            fire_scatter(m)
        return carry
    lax.fori_loop(0, GCHUNKS // 4, quad, 0)
    wait_scatter(2)
    wait_scatter(3)

    plsc.subcore_barrier()

    pltpu.sync_copy(acc_sh.at[pl.ds(r_own, ROWS_PER_SUB)],
                    part_hbm.at[cid, pl.ds(r_own, ROWS_PER_SUB)])


def _sc_scatter(h, src2d, dst2d):
    mesh = plsc.VectorSubcoreMesh(core_axis_name="c", subcore_axis_name="s")
    fn = functools.partial(
        pl.kernel,
        out_type=jax.ShapeDtypeStruct((NC, ACC_ROWS, HID), jnp.float32),
        mesh=mesh,
        scratch_types=(
            [pltpu.VMEM((1, 128), jnp.int32) for _ in range(8)]
            + [pltpu.VMEM((128, HID), jnp.float32) for _ in range(4)]
            + [pltpu.VMEM((1, 128), jnp.int32) for _ in range(4)]
            + [pltpu.VMEM((48, HID), jnp.float32)]
            + [pltpu.SemaphoreType.DMA for _ in range(12)]
            + [pltpu.VMEM_SHARED((ACC_ROWS, HID), jnp.float32)]
        ),
    )(_sc_body)
    return fn(h, src2d, dst2d)


def _sc_cnt_body(dst_hbm, cntp_hbm,
                 dst0, dst1, loc0, loc1, ones_v, zcnt_v,
                 sem_s, sem_i0, sem_i1, cnt_sh):
    cid = lax.axis_index("c")
    sid = lax.axis_index("s")

    zero16 = jnp.zeros((L,), jnp.float32)
    one16 = jnp.ones((L,), jnp.float32)
    base_node = cid * NODES_PER_CORE

    def zcrow(i, carry):
        for j in range(128 // L):
            zcnt_v[i, pl.ds(j * L, L)] = zero16
        return carry
    lax.fori_loop(0, 48, zcrow, 0)

    def orow(i, carry):
        for j in range(128 // L):
            ones_v[i, pl.ds(j * L, L)] = one16
        return carry
    lax.fori_loop(0, 128, orow, 0)

    r_own = sid * ROWS_PER_SUB

    def zc(i, carry):
        pltpu.sync_copy(zcnt_v, cnt_sh.at[pl.ds(r_own + i * 48, 48)])
        return carry
    lax.fori_loop(0, ROWS_PER_SUB // 48, zc, 0)

    plsc.subcore_barrier()

    base = sid * ROWS_PER_WORKER
    dsts = (dst0, dst1)
    locs = (loc0, loc1)
    sems = (sem_i0, sem_i1)

    def fire_idx(c, cur):
        pltpu.async_copy(dst_hbm.at[pl.ds(base + c * CH_ROWS, CH_ROWS)],
                         dsts[cur], sems[cur])

    def wait_idx(cur):
        pltpu.make_async_copy(dst_hbm.at[pl.ds(base, CH_ROWS)],
                              dsts[cur], sems[cur]).wait()

    def remap(cur):
        for j in range(CH_ROWS):
            for k in range(128 // L):
                d = dsts[cur][j, pl.ds(k * L, L)]
                loc = d - base_node
                bad = (loc < 0) | (loc >= NODES_PER_CORE)
                locs[cur][j, pl.ds(k * L, L)] = jnp.where(
                    bad, TRASH_ROW + (d & 127), loc)

    def fire_scatter(cur):
        for j in range(CH_ROWS):
            pltpu.async_copy(ones_v, cnt_sh.at[locs[cur].at[j]], sem_s,
                             add=True)

    def wait_scatter(cur):
        for j in range(CH_ROWS):
            pltpu.make_async_copy(ones_v, cnt_sh.at[locs[cur].at[j]],
                                  sem_s).wait()

    pltpu.sync_copy(dst_hbm.at[pl.ds(base, CH_ROWS)], dst0)
    fire_idx(1, 1)

    def pair(p, carry):
        for cur in range(2):
            c = 2 * p + cur
            other = 1 - cur
            if cur == 0:
                @pl.when(p > 0)
                def _():
                    wait_idx(0)
            else:
                wait_idx(1)
            remap(cur)
            if cur == 0:
                @pl.when(p > 0)
                def _():
                    wait_scatter(other)
            else:
                wait_scatter(other)
            fire_scatter(cur)

            @pl.when(c + 2 < CHUNKS)
            def _():
                fire_idx(c + 2, cur)
        return carry
    lax.fori_loop(0, CHUNKS // 2, pair, 0)
    wait_scatter(1)

    plsc.subcore_barrier()

    pltpu.sync_copy(cnt_sh.at[pl.ds(r_own, ROWS_PER_SUB)],
                    cntp_hbm.at[cid, pl.ds(r_own, ROWS_PER_SUB)])


def _sc_counts(dst2d):
    mesh = plsc.VectorSubcoreMesh(core_axis_name="c", subcore_axis_name="s")
    fn = functools.partial(
        pl.kernel,
        out_type=jax.ShapeDtypeStruct((NC, ACC_ROWS, 128), jnp.float32),
        mesh=mesh,
        scratch_types=[
            pltpu.VMEM((CH_ROWS, 128), jnp.int32),
            pltpu.VMEM((CH_ROWS, 128), jnp.int32),
            pltpu.VMEM((CH_ROWS, 128), jnp.int32),
            pltpu.VMEM((CH_ROWS, 128), jnp.int32),
            pltpu.VMEM((128, 128), jnp.float32),
            pltpu.VMEM((48, 128), jnp.float32),
            pltpu.SemaphoreType.DMA,
            pltpu.SemaphoreType.DMA,
            pltpu.SemaphoreType.DMA,
            pltpu.VMEM_SHARED((ACC_ROWS, 128), jnp.float32),
        ],
    )(_sc_cnt_body)
    return fn(dst2d)


def _fin_body(part_ref, cnt_ref, bias_ref, out_ref):
    lo = part_ref[0, :NODES_PER_CORE]
    hi = part_ref[1, :N_NODES - NODES_PER_CORE]
    s = jnp.concatenate([lo, hi], axis=0)
    clo = cnt_ref[0, :NODES_PER_CORE, 0:1]
    chi = cnt_ref[1, :N_NODES - NODES_PER_CORE, 0:1]
    c = jnp.maximum(jnp.concatenate([clo, chi], axis=0), 1.0)
    out_ref[...] = s / c + bias_ref[...]


def _finalize(part, cntp, conv_bias):
    return pl.pallas_call(
        _fin_body,
        out_shape=jax.ShapeDtypeStruct((N_NODES, HID), jnp.float32),
    )(part, cntp, conv_bias.reshape(1, HID))


def kernel(z, edge_index, W1, b1, W2, b2, W3, b3, conv_bias):
    src = edge_index[0]
    dst = edge_index[1]
    pad = E_PAD - N_EDGES
    src_p = jnp.concatenate([src, jnp.zeros((pad,), jnp.int32)])
    dst_p = jnp.concatenate([dst, jnp.full((pad,), N_NODES, jnp.int32)])
    src2d = src_p.reshape(R2D, 128)
    dst2d = dst_p.reshape(R2D, 128)
    cntp = _sc_counts(dst2d)
    h = _mlp(z, W1, b1, W2, b2, W3, b3)
    part = _sc_scatter(h, src2d, dst2d)
    return _finalize(part, cntp, conv_bias)


# speedup vs baseline: 2.3835x; 1.0007x over previous
import functools

import jax
import jax.numpy as jnp
from jax import lax
from jax.experimental import pallas as pl
from jax.experimental.pallas import tpu as pltpu
from jax.experimental.pallas import tpu_sc as plsc

N_NODES = 10000
N_EDGES = 320000
HID = 128
IN_CH = 2560

NC = 2
NS = 16
L = 16

CH_ROWS = 4
CHUNKS = 40
ROWS_PER_WORKER = 160
R2D = NS * ROWS_PER_WORKER
E_PAD = R2D * 128
NODES_PER_CORE = 5120
ACC_ROWS = 5376
TRASH_ROW = 5248
ROWS_PER_SUB = ACC_ROWS // NS

GCHUNKS = ROWS_PER_WORKER
GPAIRS = GCHUNKS // 2


def _mlp_body(z_ref, w1_ref, b1_ref, w2_ref, b2_ref, w3_ref, b3_ref, out_ref):
    h = jnp.dot(z_ref[...], w1_ref[...], preferred_element_type=jnp.float32)
    h = jnp.maximum(h + b1_ref[...], 0.0)
    h = jnp.dot(h, w2_ref[...], preferred_element_type=jnp.float32)
    h = jnp.maximum(h + b2_ref[...], 0.0)
    h = jnp.dot(h, w3_ref[...], preferred_element_type=jnp.float32)
    out_ref[...] = h + b3_ref[...]


def _mlp(z, W1, b1, W2, b2, W3, b3):
    B = 2000
    grid = (N_NODES // B,)
    return pl.pallas_call(
        _mlp_body,
        grid=grid,
        in_specs=[
            pl.BlockSpec((B, IN_CH), lambda i: (i, 0)),
            pl.BlockSpec((IN_CH, HID), lambda i: (0, 0)),
            pl.BlockSpec((1, HID), lambda i: (0, 0)),
            pl.BlockSpec((HID, HID), lambda i: (0, 0)),
            pl.BlockSpec((1, HID), lambda i: (0, 0)),
            pl.BlockSpec((HID, HID), lambda i: (0, 0)),
            pl.BlockSpec((1, HID), lambda i: (0, 0)),
        ],
        out_specs=pl.BlockSpec((B, HID), lambda i: (i, 0)),
        out_shape=jax.ShapeDtypeStruct((N_NODES, HID), jnp.float32),
    )(z, W1, b1.reshape(1, HID), W2, b2.reshape(1, HID), W3, b3.reshape(1, HID))


def _sc_body(h_hbm, src_hbm, dst_hbm, part_hbm,
             src0, src1, src2, src3, dst0, dst1, dst2, dst3,
             rows0, rows1, rows2, rows3, loc0, loc1, loc2, loc3, zbuf_v,
             sg0, sg1, sg2, sg3, ss0, ss1, ss2, ss3, si0, si1, si2, si3,
             acc_sh):
    cid = lax.axis_index("c")
    sid = lax.axis_index("s")

    zero16 = jnp.zeros((L,), jnp.float32)
    base_node = cid * NODES_PER_CORE

    def zrow(i, carry):
        for j in range(HID // L):
            zbuf_v[i, pl.ds(j * L, L)] = zero16
        return carry
    lax.fori_loop(0, 48, zrow, 0)

    r_own = sid * ROWS_PER_SUB

    def zacc(i, carry):
        pltpu.sync_copy(zbuf_v, acc_sh.at[pl.ds(r_own + i * 48, 48)])
        return carry
    lax.fori_loop(0, ROWS_PER_SUB // 48, zacc, 0)

    plsc.subcore_barrier()

    base = sid * ROWS_PER_WORKER
    srcs = (src0, src1, src2, src3)
    dsts = (dst0, dst1, dst2, dst3)
    rows = (rows0, rows1, rows2, rows3)
    locs = (loc0, loc1, loc2, loc3)
    sgs = (sg0, sg1, sg2, sg3)
    sss = (ss0, ss1, ss2, ss3)
    sis = (si0, si1, si2, si3)

    def fire_idx(c, m):
        pltpu.async_copy(src_hbm.at[pl.ds(base + c, 1)], srcs[m], sis[m])
        pltpu.async_copy(dst_hbm.at[pl.ds(base + c, 1)], dsts[m], sis[m])

    def wait_idx(m):
        pltpu.make_async_copy(src_hbm.at[pl.ds(base, 1)], srcs[m],
                              sis[m]).wait()
        pltpu.make_async_copy(dst_hbm.at[pl.ds(base, 1)], dsts[m],
                              sis[m]).wait()

    def fire_gather(m):
        pltpu.async_copy(h_hbm.at[srcs[m].at[0]], rows[m], sgs[m])

    def wait_gather(m):
        pltpu.make_async_copy(h_hbm.at[srcs[m].at[0]], rows[m],
                              sgs[m]).wait()

    def remap(m):
        for k in range(128 // L):
            d = dsts[m][0, pl.ds(k * L, L)]
            loc = d - base_node
            bad = (loc < 0) | (loc >= NODES_PER_CORE)
            loc = jnp.where(bad, TRASH_ROW + (d & 127), loc)
            locs[m][0, pl.ds(k * L, L)] = loc

    def fire_scatter(m):
        pltpu.async_copy(rows[m], acc_sh.at[locs[m].at[0]], sss[m],
                         add=True)

    def wait_scatter(m):
        pltpu.make_async_copy(rows[m], acc_sh.at[locs[m].at[0]],
                              sss[m]).wait()

    pltpu.sync_copy(src_hbm.at[pl.ds(base, 1)], src0)
    pltpu.sync_copy(dst_hbm.at[pl.ds(base, 1)], dst0)
    pltpu.sync_copy(src_hbm.at[pl.ds(base + 1, 1)], src1)
    pltpu.sync_copy(dst_hbm.at[pl.ds(base + 1, 1)], dst1)
    fire_idx(2, 2)
    fire_idx(3, 3)
    fire_gather(0)
    fire_gather(1)

    def quad(q, carry):
        for m in range(4):
            c = 4 * q + m
            if m < 2:
                @pl.when(q > 0)
                def _():
                    wait_scatter((m + 2) % 4)
            else:
                wait_scatter((m + 2) % 4)
            if m < 2:
                wait_idx((m + 2) % 4)
                fire_gather((m + 2) % 4)
            else:
                @pl.when(q < GCHUNKS // 4 - 1)
                def _():
                    wait_idx((m + 2) % 4)
                    fire_gather((m + 2) % 4)
            wait_gather(m)
            remap(m)

            @pl.when(q < GCHUNKS // 4 - 1)
            def _():
                fire_idx(c + 4, m)
            fire_scatter(m)
        return carry
    lax.fori_loop(0, GCHUNKS // 4, quad, 0)
    wait_scatter(2)
    wait_scatter(3)

    plsc.subcore_barrier()

    pltpu.sync_copy(acc_sh.at[pl.ds(r_own, ROWS_PER_SUB)],
                    part_hbm.at[cid, pl.ds(r_own, ROWS_PER_SUB)])


def _sc_scatter(h, src2d, dst2d):
    mesh = plsc.VectorSubcoreMesh(core_axis_name="c", subcore_axis_name="s")
    fn = functools.partial(
        pl.kernel,
        out_type=jax.ShapeDtypeStruct((NC, ACC_ROWS, HID), jnp.float32),
        mesh=mesh,
        scratch_types=(
            [pltpu.VMEM((1, 128), jnp.int32) for _ in range(8)]
            + [pltpu.VMEM((128, HID), jnp.float32) for _ in range(4)]
            + [pltpu.VMEM((1, 128), jnp.int32) for _ in range(4)]
            + [pltpu.VMEM((48, HID), jnp.float32)]
            + [pltpu.SemaphoreType.DMA for _ in range(12)]
            + [pltpu.VMEM_SHARED((ACC_ROWS, HID), jnp.float32)]
        ),
    )(_sc_body)
    return fn(h, src2d, dst2d)


def _sc_cnt_body(dst_hbm, cntp_hbm,
                 dst0, dst1, loc0, loc1, ones_v, zcnt_v,
                 sem_s, sem_i0, sem_i1, cnt_sh):
    cid = lax.axis_index("c")
    sid = lax.axis_index("s")

    zero16 = jnp.zeros((L,), jnp.float32)
    one16 = jnp.ones((L,), jnp.float32)
    base_node = cid * NODES_PER_CORE

    def zcrow(i, carry):
        for j in range(128 // L):
            zcnt_v[i, pl.ds(j * L, L)] = zero16
        return carry
    lax.fori_loop(0, 48, zcrow, 0)

    def orow(i, carry):
        for j in range(128 // L):
            ones_v[i, pl.ds(j * L, L)] = one16
        return carry
    lax.fori_loop(0, 128, orow, 0)

    r_own = sid * ROWS_PER_SUB

    def zc(i, carry):
        pltpu.sync_copy(zcnt_v, cnt_sh.at[pl.ds(r_own + i * 48, 48)])
        return carry
    lax.fori_loop(0, ROWS_PER_SUB // 48, zc, 0)

    plsc.subcore_barrier()

    base = sid * ROWS_PER_WORKER
    dsts = (dst0, dst1)
    locs = (loc0, loc1)
    sems = (sem_i0, sem_i1)

    def fire_idx(c, cur):
        pltpu.async_copy(dst_hbm.at[pl.ds(base + c * CH_ROWS, CH_ROWS)],
                         dsts[cur], sems[cur])

    def wait_idx(cur):
        pltpu.make_async_copy(dst_hbm.at[pl.ds(base, CH_ROWS)],
                              dsts[cur], sems[cur]).wait()

    def remap(cur):
        for j in range(CH_ROWS):
            for k in range(128 // L):
                d = dsts[cur][j, pl.ds(k * L, L)]
                loc = d - base_node
                bad = (loc < 0) | (loc >= NODES_PER_CORE)
                locs[cur][j, pl.ds(k * L, L)] = jnp.where(
                    bad, TRASH_ROW + (d & 127), loc)

    def fire_scatter(cur):
        for j in range(CH_ROWS):
            pltpu.async_copy(ones_v, cnt_sh.at[locs[cur].at[j]], sem_s,
                             add=True)

    def wait_scatter(cur):
        for j in range(CH_ROWS):
            pltpu.make_async_copy(ones_v, cnt_sh.at[locs[cur].at[j]],
                                  sem_s).wait()

    pltpu.sync_copy(dst_hbm.at[pl.ds(base, CH_ROWS)], dst0)
    fire_idx(1, 1)

    def pair(p, carry):
        for cur in range(2):
            c = 2 * p + cur
            other = 1 - cur
            if cur == 0:
                @pl.when(p > 0)
                def _():
                    wait_idx(0)
            else:
                wait_idx(1)
            remap(cur)
            if cur == 0:
                @pl.when(p > 0)
                def _():
                    wait_scatter(other)
            else:
                wait_scatter(other)
            fire_scatter(cur)

            @pl.when(c + 2 < CHUNKS)
            def _():
                fire_idx(c + 2, cur)
        return carry
    lax.fori_loop(0, CHUNKS // 2, pair, 0)
    wait_scatter(1)

    plsc.subcore_barrier()

    pltpu.sync_copy(cnt_sh.at[pl.ds(r_own, ROWS_PER_SUB)],
                    cntp_hbm.at[cid, pl.ds(r_own, ROWS_PER_SUB)])


def _sc_counts(dst2d):
    mesh = plsc.VectorSubcoreMesh(core_axis_name="c", subcore_axis_name="s")
    fn = functools.partial(
        pl.kernel,
        out_type=jax.ShapeDtypeStruct((NC, ACC_ROWS, 128), jnp.float32),
        mesh=mesh,
        scratch_types=[
            pltpu.VMEM((CH_ROWS, 128), jnp.int32),
            pltpu.VMEM((CH_ROWS, 128), jnp.int32),
            pltpu.VMEM((CH_ROWS, 128), jnp.int32),
            pltpu.VMEM((CH_ROWS, 128), jnp.int32),
            pltpu.VMEM((128, 128), jnp.float32),
            pltpu.VMEM((48, 128), jnp.float32),
            pltpu.SemaphoreType.DMA,
            pltpu.SemaphoreType.DMA,
            pltpu.SemaphoreType.DMA,
            pltpu.VMEM_SHARED((ACC_ROWS, 128), jnp.float32),
        ],
    )(_sc_cnt_body)
    return fn(dst2d)


def _fin_body(part_ref, cnt_ref, bias_ref, out_ref):
    lo = part_ref[0, :NODES_PER_CORE]
    hi = part_ref[1, :N_NODES - NODES_PER_CORE]
    s = jnp.concatenate([lo, hi], axis=0)
    clo = cnt_ref[0, :NODES_PER_CORE, 0:1]
    chi = cnt_ref[1, :N_NODES - NODES_PER_CORE, 0:1]
    c = jnp.maximum(jnp.concatenate([clo, chi], axis=0), 1.0)
    out_ref[...] = s / c + bias_ref[...]


def _finalize(part, cntp, conv_bias):
    return pl.pallas_call(
        _fin_body,
        out_shape=jax.ShapeDtypeStruct((N_NODES, HID), jnp.float32),
    )(part, cntp, conv_bias.reshape(1, HID))


def kernel(z, edge_index, W1, b1, W2, b2, W3, b3, conv_bias):
    src = edge_index[0]
    dst = edge_index[1]
    pad = E_PAD - N_EDGES
    src_p = jnp.concatenate([src, jnp.zeros((pad,), jnp.int32)])
    dst_p = jnp.concatenate([dst, jnp.full((pad,), N_NODES, jnp.int32)])
    src2d = src_p.reshape(R2D, 128)
    dst2d = dst_p.reshape(R2D, 128)
    cntp = _sc_counts(dst2d)
    h = _mlp(z, W1, b1, W2, b2, W3, b3)
    part = _sc_scatter(h, src2d, dst2d)
    return _finalize(part, cntp, conv_bias)
